# baseline (speedup 1.0000x reference)
import functools

import jax
import jax.numpy as jnp
from jax import lax
from jax.experimental import pallas as pl
from jax.experimental.pallas import tpu as pltpu
from jax.experimental.pallas import tpu_sc as plsc

FILTER_THRESHOLD = 0.95
FILTER_FACTOR = 0.5

_CHUNK_ROWS = 8
_UNROLL = 8


def _make_sc_kernel(B):
    info = plsc.get_sparse_core_info()
    nw = info.num_cores * info.num_subcores
    lanes = info.num_lanes
    rows_w = B // nw
    n_chunks = rows_w // _CHUNK_ROWS
    vecs_per_row = B // lanes
    mesh = plsc.VectorSubcoreMesh(core_axis_name="c", subcore_axis_name="s")

    @functools.partial(
        pl.kernel,
        mesh=mesh,
        compiler_params=pltpu.CompilerParams(use_tc_tiling_on_sc=False, needs_layout_passes=False),
        out_type=jax.ShapeDtypeStruct((B, B), jnp.float32),
        scratch_types=[
            pltpu.VMEM((_CHUNK_ROWS, B), jnp.float32),
            pltpu.VMEM((lanes,), jnp.float32),
        ],
    )
    def sc_filter(scores_hbm, out_hbm, buf, tbuf):
        wid = lax.axis_index("s") * info.num_cores + lax.axis_index("c")
        base = wid * rows_w
        lane = lax.iota(jnp.int32, lanes)
        ridx = lax.rem(lane, _CHUNK_ROWS)
        dmask = lane < _CHUNK_ROWS

        def chunk_body(g, carry):
            row0 = base + g * _CHUNK_ROWS
            pltpu.sync_copy(scores_hbm.at[pl.ds(row0, _CHUNK_ROWS)], buf)
            cidx = row0 + ridx
            diag = plsc.load_gather(buf, [ridx, cidx], mask=dmask)
            tbuf[...] = diag * FILTER_THRESHOLD

            def row_body(r, carry2):
                tvec = plsc.load_gather(tbuf, [jnp.full((lanes,), r, jnp.int32)])

                def col_body(j, carry3):
                    off = j * (lanes * _UNROLL)
                    for u in range(_UNROLL):
                        sl = pl.ds(off + u * lanes, lanes)
                        v = buf[r, sl]
                        buf[r, sl] = jnp.where(v > tvec, v * FILTER_FACTOR, v)
                    return carry3

                return lax.fori_loop(0, vecs_per_row // _UNROLL, col_body, carry2)

            lax.fori_loop(0, _CHUNK_ROWS, row_body, 0)
            plsc.store_scatter(buf, [ridx, cidx], diag, mask=dmask)
            pltpu.sync_copy(buf, out_hbm.at[pl.ds(row0, _CHUNK_ROWS)])
            return carry

        lax.fori_loop(0, n_chunks, chunk_body, 0)

    return sc_filter


def kernel(scores):
    B = scores.shape[0]
    return _make_sc_kernel(B)(scores)


# speedup vs baseline: 1.7725x; 1.7725x over previous
import functools

import jax
import jax.numpy as jnp
from jax import lax
from jax.experimental import pallas as pl
from jax.experimental.pallas import tpu as pltpu
from jax.experimental.pallas import tpu_sc as plsc

FILTER_THRESHOLD = 0.95
FILTER_FACTOR = 0.5

_CHUNK_ROWS = 8
_UNROLL = 8


def _make_sc_kernel(B):
    info = plsc.get_sparse_core_info()
    nw = info.num_cores * info.num_subcores
    lanes = info.num_lanes
    rows_w = B // nw
    n_chunks = rows_w // _CHUNK_ROWS
    vecs_per_row = B // lanes
    mesh = plsc.VectorSubcoreMesh(core_axis_name="c", subcore_axis_name="s")

    @functools.partial(
        pl.kernel,
        mesh=mesh,
        compiler_params=pltpu.CompilerParams(
            use_tc_tiling_on_sc=False, needs_layout_passes=False
        ),
        out_type=jax.ShapeDtypeStruct((B, B), jnp.float32),
        scratch_types=[
            pltpu.VMEM((_CHUNK_ROWS, B), jnp.float32),
            pltpu.VMEM((lanes,), jnp.float32),
        ],
    )
    def sc_filter(scores_hbm, out_hbm, buf, tbuf):
        wid = lax.axis_index("s") * info.num_cores + lax.axis_index("c")
        base = wid * rows_w
        lane = lax.iota(jnp.int32, lanes)
        ridx = lax.rem(lane, _CHUNK_ROWS)
        dmask = lane < _CHUNK_ROWS

        def chunk_body(g, carry):
            row0 = base + g * _CHUNK_ROWS
            pltpu.sync_copy(scores_hbm.at[pl.ds(row0, _CHUNK_ROWS)], buf)
            cidx = row0 + ridx
            diag = plsc.load_gather(buf, [ridx, cidx], mask=dmask)
            th = diag * FILTER_THRESHOLD

            for r in range(_CHUNK_ROWS):
                tvec = lax.gather(
                    th,
                    jnp.full((lanes, 1), r, jnp.int32),
                    lax.GatherDimensionNumbers(
                        offset_dims=(),
                        collapsed_slice_dims=(0,),
                        start_index_map=(0,),
                    ),
                    (1,),
                    mode=lax.GatherScatterMode.PROMISE_IN_BOUNDS,
                )

                def col_body(j, carry3, r=r, tvec=tvec):
                    off = j * (lanes * _UNROLL)
                    for u in range(_UNROLL):
                        sl = pl.ds(off + u * lanes, lanes)
                        v = buf[r, sl]
                        buf[r, sl] = jnp.where(v > tvec, v * FILTER_FACTOR, v)
                    return carry3

                lax.fori_loop(0, vecs_per_row // _UNROLL, col_body, 0)

            plsc.store_scatter(buf, [ridx, cidx], diag, mask=dmask)
            pltpu.sync_copy(buf, out_hbm.at[pl.ds(row0, _CHUNK_ROWS)])
            return carry

        lax.fori_loop(0, n_chunks, chunk_body, 0)

    return sc_filter


def kernel(scores):
    B = scores.shape[0]
    return _make_sc_kernel(B)(scores)


# speedup vs baseline: 1.7864x; 1.0079x over previous
import functools

import jax
import jax.numpy as jnp
from jax import lax
from jax.experimental import pallas as pl
from jax.experimental.pallas import tpu as pltpu
from jax.experimental.pallas import tpu_sc as plsc

FILTER_THRESHOLD = 0.95
FILTER_FACTOR = 0.5

_CHUNK_ROWS = 8
_UNROLL = 8


def _make_sc_kernel(B):
    info = plsc.get_sparse_core_info()
    nw = info.num_cores * info.num_subcores
    lanes = info.num_lanes
    rows_w = B // nw
    n_chunks = rows_w // _CHUNK_ROWS
    vecs_per_row = B // lanes
    mesh = plsc.VectorSubcoreMesh(core_axis_name="c", subcore_axis_name="s")

    @functools.partial(
        pl.kernel,
        mesh=mesh,
        compiler_params=pltpu.CompilerParams(
            use_tc_tiling_on_sc=False, needs_layout_passes=False
        ),
        out_type=jax.ShapeDtypeStruct((B, B), jnp.float32),
        scratch_types=[
            pltpu.VMEM((_CHUNK_ROWS, B), jnp.float32),
            pltpu.VMEM((lanes,), jnp.float32),
        ],
    )
    def sc_filter(scores_hbm, out_hbm, buf, tbuf):
        wid = lax.axis_index("s") * info.num_cores + lax.axis_index("c")
        base = wid * rows_w
        lane = lax.iota(jnp.int32, lanes)
        ridx = lax.rem(lane, _CHUNK_ROWS)
        dmask = lane < _CHUNK_ROWS

        def chunk_body(g, carry):
            row0 = base + g * _CHUNK_ROWS
            pltpu.sync_copy(scores_hbm.at[pl.ds(row0, _CHUNK_ROWS)], buf)
            cidx = row0 + ridx
            diag = plsc.load_gather(buf, [ridx, cidx], mask=dmask)
            th = diag * FILTER_THRESHOLD

            for r in range(_CHUNK_ROWS):
                tvec = lax.gather(
                    th,
                    jnp.full((lanes, 1), r, jnp.int32),
                    lax.GatherDimensionNumbers(
                        offset_dims=(),
                        collapsed_slice_dims=(0,),
                        start_index_map=(0,),
                    ),
                    (1,),
                    mode=lax.GatherScatterMode.PROMISE_IN_BOUNDS,
                )

                @plsc.parallel_loop(0, B, step=lanes, unroll=_UNROLL)
                def col_body(c, r=r, tvec=tvec):
                    sl = pl.ds(c, lanes)
                    v = buf[r, sl]
                    buf[r, sl] = jnp.where(v > tvec, v * FILTER_FACTOR, v)

            plsc.store_scatter(buf, [ridx, cidx], diag, mask=dmask)
            pltpu.sync_copy(buf, out_hbm.at[pl.ds(row0, _CHUNK_ROWS)])
            return carry

        lax.fori_loop(0, n_chunks, chunk_body, 0)

    return sc_filter


def kernel(scores):
    B = scores.shape[0]
    return _make_sc_kernel(B)(scores)


# speedup vs baseline: 3.2816x; 1.8369x over previous
import functools

import jax
import jax.numpy as jnp
from jax import lax
from jax.experimental import pallas as pl
from jax.experimental.pallas import tpu as pltpu
from jax.experimental.pallas import tpu_sc as plsc

FILTER_THRESHOLD = 0.95
FILTER_FACTOR = 0.5

_CHUNK_ROWS = 8
_UNROLL = 8

_BCAST_DNUMS = lax.GatherDimensionNumbers(
    offset_dims=(), collapsed_slice_dims=(0,), start_index_map=(0,)
)


def _lane_broadcast(vec, idx, lanes):
    return lax.gather(
        vec,
        jnp.full((lanes, 1), idx, jnp.int32),
        _BCAST_DNUMS,
        (1,),
        mode=lax.GatherScatterMode.PROMISE_IN_BOUNDS,
    )


def _make_sc_kernel(B):
    info = plsc.get_sparse_core_info()
    nw = info.num_cores * info.num_subcores
    lanes = info.num_lanes
    rows_w = B // nw
    n_chunks = rows_w // _CHUNK_ROWS
    mesh = plsc.VectorSubcoreMesh(core_axis_name="c", subcore_axis_name="s")

    @functools.partial(
        pl.kernel,
        mesh=mesh,
        compiler_params=pltpu.CompilerParams(use_tc_tiling_on_sc=True),
        out_type=jax.ShapeDtypeStruct((B, B), jnp.float32),
        scratch_types=[
            pltpu.VMEM((_CHUNK_ROWS, B), jnp.float32),
        ],
    )
    def sc_filter(scores_hbm, out_hbm, buf):
        wid = lax.axis_index("s") * info.num_cores + lax.axis_index("c")
        base = wid * rows_w
        lane = lax.iota(jnp.int32, lanes)

        def chunk_body(g, carry):
            row0 = base + g * _CHUNK_ROWS
            pltpu.sync_copy(scores_hbm.at[pl.ds(row0, _CHUNK_ROWS)], buf)

            for r in range(_CHUNK_ROWS):
                dcol = row0 + r
                align = (dcol // lanes) * lanes
                dvec = buf[r, pl.ds(align, lanes)]
                th = _lane_broadcast(dvec, dcol - align, lanes) * FILTER_THRESHOLD

                @plsc.parallel_loop(0, B, step=lanes, unroll=_UNROLL)
                def col_body(c, r=r, th=th, dcol=dcol):
                    sl = pl.ds(c, lanes)
                    v = buf[r, sl]
                    m = (v > th) & ((c + lane) != dcol)
                    buf[r, sl] = jnp.where(m, v * FILTER_FACTOR, v)

            pltpu.sync_copy(buf, out_hbm.at[pl.ds(row0, _CHUNK_ROWS)])
            return carry

        lax.fori_loop(0, n_chunks, chunk_body, 0)

    return sc_filter


def kernel(scores):
    B = scores.shape[0]
    return _make_sc_kernel(B)(scores)


# speedup vs baseline: 5.4527x; 1.6616x over previous
import functools

import jax
import jax.numpy as jnp
from jax import lax
from jax.experimental import pallas as pl
from jax.experimental.pallas import tpu as pltpu
from jax.experimental.pallas import tpu_sc as plsc

FILTER_THRESHOLD = 0.95
FILTER_FACTOR = 0.5

_CHUNK_ROWS = 8
_UNROLL = 8
_NBUF = 3

_BCAST_DNUMS = lax.GatherDimensionNumbers(
    offset_dims=(), collapsed_slice_dims=(0,), start_index_map=(0,)
)


def _lane_broadcast(vec, idx, lanes):
    return lax.gather(
        vec,
        jnp.full((lanes, 1), idx, jnp.int32),
        _BCAST_DNUMS,
        (1,),
        mode=lax.GatherScatterMode.PROMISE_IN_BOUNDS,
    )


def _make_sc_kernel(B):
    info = plsc.get_sparse_core_info()
    nw = info.num_cores * info.num_subcores
    lanes = info.num_lanes
    rows_w = B // nw
    n_chunks = rows_w // _CHUNK_ROWS
    mesh = plsc.VectorSubcoreMesh(core_axis_name="c", subcore_axis_name="s")

    @functools.partial(
        pl.kernel,
        mesh=mesh,
        compiler_params=pltpu.CompilerParams(use_tc_tiling_on_sc=True),
        out_type=jax.ShapeDtypeStruct((B, B), jnp.float32),
        scratch_types=[
            pltpu.VMEM((_NBUF, _CHUNK_ROWS, B), jnp.float32),
            pltpu.SemaphoreType.DMA((_NBUF,)),
            pltpu.SemaphoreType.DMA((_NBUF,)),
        ],
    )
    def sc_filter(scores_hbm, out_hbm, bufs, sem_in, sem_out):
        wid = lax.axis_index("s") * info.num_cores + lax.axis_index("c")
        base = wid * rows_w
        lane = lax.iota(jnp.int32, lanes)

        def start_in(g):
            row0 = base + g * _CHUNK_ROWS
            pltpu.async_copy(
                scores_hbm.at[pl.ds(row0, _CHUNK_ROWS)],
                bufs.at[g % _NBUF],
                sem_in.at[g % _NBUF],
            )

        def start_out(g):
            row0 = base + g * _CHUNK_ROWS
            pltpu.async_copy(
                bufs.at[g % _NBUF],
                out_hbm.at[pl.ds(row0, _CHUNK_ROWS)],
                sem_out.at[g % _NBUF],
            )

        def wait_out(g):
            row0 = base + g * _CHUNK_ROWS
            pltpu.make_async_copy(
                bufs.at[g % _NBUF],
                out_hbm.at[pl.ds(row0, _CHUNK_ROWS)],
                sem_out.at[g % _NBUF],
            ).wait()

        def wait_in(g):
            row0 = base + g * _CHUNK_ROWS
            pltpu.make_async_copy(
                scores_hbm.at[pl.ds(row0, _CHUNK_ROWS)],
                bufs.at[g % _NBUF],
                sem_in.at[g % _NBUF],
            ).wait()

        start_in(0)

        def chunk_body(g, carry):
            @pl.when(g >= _NBUF - 1)
            def _():
                wait_out(g - (_NBUF - 1))

            @pl.when(g + 1 < n_chunks)
            def _():
                start_in(g + 1)

            wait_in(g)
            row0 = base + g * _CHUNK_ROWS
            b = g % _NBUF

            for r in range(_CHUNK_ROWS):
                dcol = row0 + r
                align = (dcol // lanes) * lanes
                dvec = bufs[b, r, pl.ds(align, lanes)]
                th = _lane_broadcast(dvec, dcol - align, lanes) * FILTER_THRESHOLD

                @plsc.parallel_loop(0, B, step=lanes, unroll=_UNROLL)
                def col_body(c, b=b, r=r, th=th, dcol=dcol):
                    sl = pl.ds(c, lanes)
                    v = bufs[b, r, sl]
                    m = (v > th) & ((c + lane) != dcol)
                    bufs[b, r, sl] = jnp.where(m, v * FILTER_FACTOR, v)

            start_out(g)
            return carry

        lax.fori_loop(0, n_chunks, chunk_body, 0)
        for t in range(_NBUF - 1):
            wait_out(n_chunks - (_NBUF - 1) + t)

    return sc_filter


def kernel(scores):
    B = scores.shape[0]
    return _make_sc_kernel(B)(scores)
